# baseline (speedup 1.0000x reference)
import jax
import jax.numpy as jnp
from jax import lax
from jax.experimental import pallas as pl
from jax.experimental.pallas import tpu as pltpu


def _ceil_to(n, m):
    return ((n + m - 1) // m) * m


_TILE = 1024


def _fwd_body(x_ref, t1_ref, b1_ref, t2_ref, b2_ref,
              w1_ref, fb1_ref, w2_ref, fb2_ref, w3_ref, fb3_ref, o_ref):
    q1 = t1_ref.shape[1] // 4
    q2 = t2_ref.shape[1] // 4

    xc = x_ref[...].astype(jnp.bfloat16).reshape(784, _TILE)

    d = lax.dot_general(xc, t1_ref[...], (((0,), (0,)), ((), ())),
                        preferred_element_type=jnp.float32)
    m = jnp.maximum(jnp.maximum(d[:, :q1], d[:, q1:2 * q1]),
                    jnp.maximum(d[:, 2 * q1:3 * q1], d[:, 3 * q1:]))
    h = jnp.maximum(m + b1_ref[...], 0.0).astype(jnp.bfloat16)

    e = jnp.dot(h, t2_ref[...], preferred_element_type=jnp.float32)
    m2 = jnp.maximum(jnp.maximum(e[:, :q2], e[:, q2:2 * q2]),
                     jnp.maximum(e[:, 2 * q2:3 * q2], e[:, 3 * q2:]))
    g = jnp.maximum(m2 + b2_ref[...], 0.0).astype(jnp.bfloat16)

    z = jnp.dot(g, w1_ref[...], preferred_element_type=jnp.float32)
    z = jnp.maximum(z + fb1_ref[...], 0.0).astype(jnp.bfloat16)
    z = jnp.dot(z, w2_ref[...], preferred_element_type=jnp.float32)
    z = jnp.maximum(z + fb2_ref[...], 0.0).astype(jnp.bfloat16)
    o = jnp.dot(z, w3_ref[...], preferred_element_type=jnp.float32)
    o_ref[...] = o + fb3_ref[...]


def kernel(x, t1, b1, t2, b2, w1, fb1, w2, fb2, w3, fb3):
    N = x.shape[0]
    padded = _ceil_to(N, _TILE)
    x2 = x.reshape(N, 784)
    if padded != N:
        x2 = jnp.pad(x2, ((0, padded - N), (0, 0)))
    xv = x2.T.reshape(784, padded // 128, 128)

    ncp = fb3.shape[-1]
    const = lambda a: pl.BlockSpec(a.shape, (lambda i: (0,) * a.ndim),
                                   pipeline_mode=pl.Buffered(1))

    out = pl.pallas_call(
        _fwd_body,
        out_shape=jax.ShapeDtypeStruct((padded, ncp), jnp.float32),
        grid=(padded // _TILE,),
        in_specs=[
            pl.BlockSpec((784, _TILE // 128, 128), lambda i: (0, i, 0)),
            const(t1), const(b1), const(t2), const(b2),
            const(w1), const(fb1), const(w2), const(fb2),
            const(w3), const(fb3),
        ],
        out_specs=pl.BlockSpec((_TILE, ncp), lambda i: (i, 0)),
        compiler_params=pltpu.CompilerParams(
            dimension_semantics=("parallel",),
            vmem_limit_bytes=56 * 1024 * 1024,
        ),
    )(xv, t1, b1, t2, b2, w1, fb1, w2, fb2, w3, fb3)
    return out[:N, :10]


# speedup vs baseline: 2.2065x; 2.2065x over previous
import jax
import jax.numpy as jnp
from jax import lax
from jax.experimental import pallas as pl
from jax.experimental.pallas import tpu as pltpu


def _ceil_to(n, m):
    return ((n + m - 1) // m) * m


_TILE = 1024


def _fwd_body(x_ref, t1_ref, b1_ref, t2_ref, b2_ref,
              w1_ref, fb1_ref, w2_ref, fb2_ref, w3_ref, fb3_ref, o_ref):
    q1 = t1_ref.shape[1] // 4
    q2 = t2_ref.shape[1] // 4

    xc = x_ref[...].astype(jnp.bfloat16).reshape(784, _TILE)

    d = lax.dot_general(xc, t1_ref[...], (((0,), (0,)), ((), ())),
                        preferred_element_type=jnp.float32)
    m = jnp.maximum(jnp.maximum(d[:, :q1], d[:, q1:2 * q1]),
                    jnp.maximum(d[:, 2 * q1:3 * q1], d[:, 3 * q1:]))
    h = jnp.maximum(m + b1_ref[...], 0.0).astype(jnp.bfloat16)

    e = jnp.dot(h, t2_ref[...], preferred_element_type=jnp.float32)
    m2 = jnp.maximum(jnp.maximum(e[:, :q2], e[:, q2:2 * q2]),
                     jnp.maximum(e[:, 2 * q2:3 * q2], e[:, 3 * q2:]))
    g = jnp.maximum(m2 + b2_ref[...], 0.0).astype(jnp.bfloat16)

    z = jnp.dot(g, w1_ref[...], preferred_element_type=jnp.float32)
    z = jnp.maximum(z + fb1_ref[...], 0.0).astype(jnp.bfloat16)
    z = jnp.dot(z, w2_ref[...], preferred_element_type=jnp.float32)
    z = jnp.maximum(z + fb2_ref[...], 0.0).astype(jnp.bfloat16)
    o = jnp.dot(z, w3_ref[...], preferred_element_type=jnp.float32)
    o_ref[...] = o + fb3_ref[...]


def kernel(x, t1, b1, t2, b2, w1, fb1, w2, fb2, w3, fb3):
    N = x.shape[0]
    padded = _ceil_to(N, _TILE)
    if padded != N:
        x = jnp.pad(x, ((0, padded - N), (0, 0), (0, 0), (0, 0)))
    xv = lax.reshape(x, (784, padded // 128, 128), dimensions=(2, 3, 1, 0))

    ncp = fb3.shape[-1]
    const = lambda a: pl.BlockSpec(a.shape, (lambda i: (0,) * a.ndim),
                                   pipeline_mode=pl.Buffered(1))

    out = pl.pallas_call(
        _fwd_body,
        out_shape=jax.ShapeDtypeStruct((padded, ncp), jnp.float32),
        grid=(padded // _TILE,),
        in_specs=[
            pl.BlockSpec((784, _TILE // 128, 128), lambda i: (0, i, 0)),
            const(t1), const(b1), const(t2), const(b2),
            const(w1), const(fb1), const(w2), const(fb2),
            const(w3), const(fb3),
        ],
        out_specs=pl.BlockSpec((_TILE, ncp), lambda i: (i, 0)),
        compiler_params=pltpu.CompilerParams(
            dimension_semantics=("parallel",),
            vmem_limit_bytes=56 * 1024 * 1024,
        ),
    )(xv, t1, b1, t2, b2, w1, fb1, w2, fb2, w3, fb3)
    return out[:N, :10]


# speedup vs baseline: 2.2908x; 1.0382x over previous
import jax
import jax.numpy as jnp
from jax import lax
from jax.experimental import pallas as pl
from jax.experimental.pallas import tpu as pltpu


def _ceil_to(n, m):
    return ((n + m - 1) // m) * m


_TILE = 1024


def _fwd_body(x_ref, t1_ref, b1_ref, t2_ref, b2_ref,
              w1_ref, fb1_ref, w2_ref, fb2_ref, w3_ref, fb3_ref, o_ref):
    q1 = t1_ref.shape[1] // 4
    q2 = t2_ref.shape[1] // 4

    xc = x_ref[...].astype(jnp.bfloat16).reshape(784, _TILE)

    d = lax.dot_general(xc, t1_ref[...], (((0,), (0,)), ((), ())),
                        preferred_element_type=jnp.float32)
    m = jnp.maximum(jnp.maximum(d[:, :q1], d[:, q1:2 * q1]),
                    jnp.maximum(d[:, 2 * q1:3 * q1], d[:, 3 * q1:]))
    h = jnp.maximum(m + b1_ref[...], 0.0).astype(jnp.bfloat16)

    e = jnp.dot(h, t2_ref[...], preferred_element_type=jnp.float32)
    m2 = jnp.maximum(jnp.maximum(e[:, :q2], e[:, q2:2 * q2]),
                     jnp.maximum(e[:, 2 * q2:3 * q2], e[:, 3 * q2:]))
    g = jnp.maximum(m2 + b2_ref[...], 0.0).astype(jnp.bfloat16)

    z = jnp.dot(g, w1_ref[...], preferred_element_type=jnp.float32)
    z = jnp.maximum(z + fb1_ref[...], 0.0).astype(jnp.bfloat16)
    z = jnp.dot(z, w2_ref[...], preferred_element_type=jnp.float32)
    z = jnp.maximum(z + fb2_ref[...], 0.0).astype(jnp.bfloat16)
    ot = lax.dot_general(w3_ref[...], z, (((0,), (1,)), ((), ())),
                         preferred_element_type=jnp.float32)
    fb3c = jnp.transpose(fb3_ref[...], (1, 0))
    nc = o_ref.shape[0]
    o_ref[...] = (ot + fb3c)[:nc, :]


def kernel(x, t1, b1, t2, b2, w1, fb1, w2, fb2, w3, fb3):
    N = x.shape[0]
    padded = _ceil_to(N, _TILE)
    if padded != N:
        x = jnp.pad(x, ((0, padded - N), (0, 0), (0, 0), (0, 0)))
    xv = lax.reshape(x, (784, padded // 128, 128), dimensions=(2, 3, 1, 0))

    const = lambda a: pl.BlockSpec(a.shape, (lambda i: (0,) * a.ndim),
                                   pipeline_mode=pl.Buffered(1))

    out = pl.pallas_call(
        _fwd_body,
        out_shape=jax.ShapeDtypeStruct((10, padded), jnp.float32),
        grid=(padded // _TILE,),
        in_specs=[
            pl.BlockSpec((784, _TILE // 128, 128), lambda i: (0, i, 0)),
            const(t1), const(b1), const(t2), const(b2),
            const(w1), const(fb1), const(w2), const(fb2),
            const(w3), const(fb3),
        ],
        out_specs=pl.BlockSpec((10, _TILE), lambda i: (0, i)),
        compiler_params=pltpu.CompilerParams(
            dimension_semantics=("parallel",),
            vmem_limit_bytes=56 * 1024 * 1024,
        ),
    )(xv, t1, b1, t2, b2, w1, fb1, w2, fb2, w3, fb3)
    return jnp.transpose(out, (1, 0))[:N]


# speedup vs baseline: 2.3143x; 1.0103x over previous
import jax
import jax.numpy as jnp
from jax import lax
from jax.experimental import pallas as pl
from jax.experimental.pallas import tpu as pltpu


def _ceil_to(n, m):
    return ((n + m - 1) // m) * m


_TILE = 2048


def _fwd_body(x_ref, t1_ref, b1_ref, t2_ref, b2_ref,
              w1_ref, fb1_ref, w2_ref, fb2_ref, w3_ref, fb3_ref, o_ref):
    q1 = t1_ref.shape[1] // 4
    q2 = t2_ref.shape[1] // 4

    xc = x_ref[...].astype(jnp.bfloat16).reshape(784, _TILE)

    dw = lambda w: lax.dot_general(
        xc, t1_ref[:, w * q1:(w + 1) * q1], (((0,), (0,)), ((), ())),
        preferred_element_type=jnp.float32)
    m = jnp.maximum(jnp.maximum(dw(0), dw(1)), jnp.maximum(dw(2), dw(3)))
    h = jnp.maximum(m + b1_ref[...], 0.0).astype(jnp.bfloat16)

    e = jnp.dot(h, t2_ref[...], preferred_element_type=jnp.float32)
    m2 = jnp.maximum(jnp.maximum(e[:, :q2], e[:, q2:2 * q2]),
                     jnp.maximum(e[:, 2 * q2:3 * q2], e[:, 3 * q2:]))
    g = jnp.maximum(m2 + b2_ref[...], 0.0).astype(jnp.bfloat16)

    z = jnp.dot(g, w1_ref[...], preferred_element_type=jnp.float32)
    z = jnp.maximum(z + fb1_ref[...], 0.0).astype(jnp.bfloat16)
    z = jnp.dot(z, w2_ref[...], preferred_element_type=jnp.float32)
    z = jnp.maximum(z + fb2_ref[...], 0.0).astype(jnp.bfloat16)
    ot = lax.dot_general(w3_ref[...], z, (((0,), (1,)), ((), ())),
                         preferred_element_type=jnp.float32)
    fb3c = jnp.transpose(fb3_ref[...], (1, 0))
    nc = o_ref.shape[0]
    o_ref[...] = (ot + fb3c)[:nc, :]


def kernel(x, t1, b1, t2, b2, w1, fb1, w2, fb2, w3, fb3):
    N = x.shape[0]
    padded = _ceil_to(N, _TILE)
    if padded != N:
        x = jnp.pad(x, ((0, padded - N), (0, 0), (0, 0), (0, 0)))
    xv = lax.reshape(x, (784, padded // 128, 128), dimensions=(2, 3, 1, 0))

    const = lambda a: pl.BlockSpec(a.shape, (lambda i: (0,) * a.ndim),
                                   pipeline_mode=pl.Buffered(1))

    out = pl.pallas_call(
        _fwd_body,
        out_shape=jax.ShapeDtypeStruct((10, padded), jnp.float32),
        grid=(padded // _TILE,),
        in_specs=[
            pl.BlockSpec((784, _TILE // 128, 128), lambda i: (0, i, 0)),
            const(t1), const(b1), const(t2), const(b2),
            const(w1), const(fb1), const(w2), const(fb2),
            const(w3), const(fb3),
        ],
        out_specs=pl.BlockSpec((10, _TILE), lambda i: (0, i)),
        compiler_params=pltpu.CompilerParams(
            dimension_semantics=("parallel",),
            vmem_limit_bytes=56 * 1024 * 1024,
        ),
    )(xv, t1, b1, t2, b2, w1, fb1, w2, fb2, w3, fb3)
    return jnp.transpose(out, (1, 0))[:N]
